# baseline (speedup 1.0000x reference)
import jax
import jax.numpy as jnp
from jax import lax
from jax.experimental import pallas as pl
from jax.experimental.pallas import tpu as pltpu
from jax.experimental.pallas import tpu_sc as plsc

N = 10000
E = 320000
D = 128
H = 8
C = 16
HID = H * C
G = 64

NC = 2
NS = 16
L = 16
NW = NC * NS
EPW = E // NW
CH = 40
NCHUNK = EPW // CH
IB = 50
NBLK = NCHUNK // IB
CH2 = 1000
NCHUNK2 = EPW // CH2
ROWS_PT = 1000
NTILE_IO = N // ROWS_PT


def _proj_body(x_ref, wq, bq, wk, bk, wv, bv, ws, bs, q_ref, k_ref, v_ref,
               s_ref):
    xb = x_ref[...]
    q = (jnp.dot(xb, wq[...], preferred_element_type=jnp.float32) + bq[...])
    q_ref[...] = q * 0.25
    k_ref[...] = jnp.dot(xb, wk[...], preferred_element_type=jnp.float32) + bk[...]
    v_ref[...] = jnp.dot(xb, wv[...], preferred_element_type=jnp.float32) + bv[...]
    s_ref[...] = jnp.dot(xb, ws[...], preferred_element_type=jnp.float32) + bs[...]


_ROWS_BLK = 1000


def _proj(x, Wq, bq, Wk, bk, Wv, bv, Ws, bs):
    full = lambda shape: pl.BlockSpec(shape, lambda i: (0, 0))
    return pl.pallas_call(
        _proj_body,
        grid=(N // _ROWS_BLK,),
        in_specs=[
            pl.BlockSpec((_ROWS_BLK, D), lambda i: (i, 0)),
            full((D, HID)), full((1, HID)),
            full((D, HID)), full((1, HID)),
            full((D, HID)), full((1, HID)),
            full((D, HID)), full((1, HID)),
        ],
        out_specs=[
            pl.BlockSpec((_ROWS_BLK, HID), lambda i: (i, 0)),
            pl.BlockSpec((_ROWS_BLK, HID), lambda i: (i, 0)),
            pl.BlockSpec((_ROWS_BLK, HID), lambda i: (i, 0)),
            pl.BlockSpec((_ROWS_BLK, HID), lambda i: (i, 0)),
        ],
        out_shape=[
            jax.ShapeDtypeStruct((N, HID), jnp.float32),
            jax.ShapeDtypeStruct((N, HID), jnp.float32),
            jax.ShapeDtypeStruct((N, HID), jnp.float32),
            jax.ShapeDtypeStruct((N, HID), jnp.float32),
        ],
    )(x, Wq, bq, Wk, bk, Wv, bv, Ws, bs)


def _pass1_body(q_hbm, k_hbm, v_hbm, src_hbm, dst_hbm, znum, zden,
                num_out, den_out, ex_out,
                src_all, dst_all, krows0, vrows0, qrows0, exbuf0,
                krows1, vrows1, qrows1, exbuf1,
                accn, accd, gsem0, gsem1):
    cid = lax.axis_index("c")
    sid = lax.axis_index("s")
    wid = cid * NS + sid
    r0 = sid * ROWS_PT
    @pl.when(sid < NTILE_IO)
    def _zero():
        pltpu.sync_copy(znum, accn.at[pl.ds(r0, ROWS_PT)])
        pltpu.sync_copy(zden, accd.at[pl.ds(r0, ROWS_PT)])
    plsc.subcore_barrier()
    iota = lax.iota(jnp.int32, L)

    bufs = ((krows0, vrows0, qrows0, exbuf0, gsem0),
            (krows1, vrows1, qrows1, exbuf1, gsem1))

    def stage_a(lc, b):
        krows, vrows, qrows, _, gsem = bufs[b]
        pltpu.async_copy(k_hbm.at[src_all.at[lc]], krows, gsem)
        pltpu.async_copy(v_hbm.at[src_all.at[lc]], vrows, gsem)
        pltpu.async_copy(q_hbm.at[dst_all.at[lc]], qrows, gsem)

    def stage_b(lc, cbase, b):
        krows, vrows, qrows, exbuf, gsem = bufs[b]
        pltpu.make_async_copy(k_hbm.at[src_all.at[lc]], krows, gsem).wait()
        pltpu.make_async_copy(v_hbm.at[src_all.at[lc]], vrows, gsem).wait()
        pltpu.make_async_copy(q_hbm.at[dst_all.at[lc]], qrows, gsem).wait()

        @plsc.parallel_loop(0, CH, unroll=8)
        def edge_body(e):
            ev = jnp.broadcast_to(e, (L,)).astype(jnp.int32)
            lane0 = iota == 0
            for h in range(H):
                qb = qrows[e, pl.ds(h * C, C)]
                kb = krows[e, pl.ds(h * C, C)]
                tot = jnp.sum(qb * kb)
                exh = jnp.exp(jnp.broadcast_to(tot, (L,)))
                vb = vrows[e, pl.ds(h * C, C)]
                vrows[e, pl.ds(h * C, C)] = exh * vb
                plsc.store_scatter(
                    exbuf, [ev, jnp.full((L,), h, jnp.int32)], exh,
                    mask=lane0)
        pltpu.sync_copy(vrows, accn.at[dst_all.at[lc]], add=True)
        pltpu.sync_copy(exbuf, accd.at[dst_all.at[lc]], add=True)
        pltpu.sync_copy(
            exbuf, ex_out.at[pl.ds(wid * EPW + (cbase + lc) * CH, CH)])

    def blk_body(ib, carry):
        cbase = ib * IB
        pltpu.sync_copy(src_hbm.at[wid, pl.ds(cbase, IB)], src_all)
        pltpu.sync_copy(dst_hbm.at[wid, pl.ds(cbase, IB)], dst_all)
        stage_a(0, 0)

        def pair_body(p, carry2):
            lc0 = 2 * p
            stage_a(lc0 + 1, 1)
            stage_b(lc0, cbase, 0)

            @pl.when(lc0 + 2 < IB)
            def _():
                stage_a(lc0 + 2, 0)
            stage_b(lc0 + 1, cbase, 1)
            return carry2

        lax.fori_loop(0, IB // 2, pair_body, 0)
        return carry

    lax.fori_loop(0, NBLK, blk_body, 0)
    plsc.subcore_barrier()

    @pl.when(sid < NTILE_IO)
    def _writeout():
        pltpu.sync_copy(accn.at[pl.ds(r0, ROWS_PT)],
                        num_out.at[cid, pl.ds(r0, ROWS_PT)])
        pltpu.sync_copy(accd.at[pl.ds(r0, ROWS_PT)],
                        den_out.at[cid, pl.ds(r0, ROWS_PT)])


def _pass1(q, k, v, src, dst):
    znum = jnp.zeros((ROWS_PT, HID), jnp.float32)
    zden = jnp.zeros((ROWS_PT, H), jnp.float32)
    mesh = plsc.VectorSubcoreMesh(
        core_axis_name="c", subcore_axis_name="s",
        num_cores=NC, num_subcores=NS)
    f = pl.kernel(
        _pass1_body,
        out_type=(
            jax.ShapeDtypeStruct((NC, N, HID), jnp.float32),
            jax.ShapeDtypeStruct((NC, N, H), jnp.float32),
            jax.ShapeDtypeStruct((E, H), jnp.float32),
        ),
        mesh=mesh,
        scratch_types=[
            pltpu.VMEM((IB, CH), jnp.int32),
            pltpu.VMEM((IB, CH), jnp.int32),
            pltpu.VMEM((CH, HID), jnp.float32),
            pltpu.VMEM((CH, HID), jnp.float32),
            pltpu.VMEM((CH, HID), jnp.float32),
            pltpu.VMEM((CH, H), jnp.float32),
            pltpu.VMEM((CH, HID), jnp.float32),
            pltpu.VMEM((CH, HID), jnp.float32),
            pltpu.VMEM((CH, HID), jnp.float32),
            pltpu.VMEM((CH, H), jnp.float32),
            pltpu.VMEM_SHARED((N, HID), jnp.float32),
            pltpu.VMEM_SHARED((N, H), jnp.float32),
            pltpu.SemaphoreType.DMA,
            pltpu.SemaphoreType.DMA,
        ],
        compiler_params=pltpu.CompilerParams(needs_layout_passes=False, use_tc_tiling_on_sc=False),
    )
    return f(q, k, v, src.reshape(NW, NCHUNK, CH), dst.reshape(NW, NCHUNK, CH),
             znum, zden)


def _fin_body(num_ref, den_ref, skip_ref, batch_ref, lnw, lnb, out_ref, inv_ref):
    num = num_ref[0] + num_ref[1]
    den = den_ref[0] + den_ref[1]
    inv = 1.0 / (den + 1e-16)
    inv_ref[...] = inv
    hh = lax.broadcasted_iota(jnp.int32, (H, HID), 0)
    jj = lax.broadcasted_iota(jnp.int32, (H, HID), 1)
    expand = (jj // C == hh).astype(jnp.float32)
    inv128 = jnp.dot(inv, expand, preferred_element_type=jnp.float32)
    out1 = num * inv128 + skip_ref[...]
    b = batch_ref[...]
    oh = (b == lax.broadcasted_iota(jnp.int32, (N, G), 1)).astype(jnp.float32)
    s = jnp.sum(out1, axis=1, keepdims=True)
    cnt = jnp.sum(oh, axis=0)
    norm = jnp.clip(cnt, 1.0, None) * HID
    dn = (((0,), (0,)), ((), ()))
    sums = lax.dot_general(oh, s, dn, preferred_element_type=jnp.float32)
    mean = sums / norm[:, None]
    mean_n = jnp.dot(oh, mean, preferred_element_type=jnp.float32)
    cen = out1 - mean_n
    s2 = jnp.sum(cen * cen, axis=1, keepdims=True)
    v2 = lax.dot_general(oh, s2, dn, preferred_element_type=jnp.float32)
    rstd = 1.0 / jnp.sqrt(v2 / norm[:, None] + 1e-5)
    rstd_n = jnp.dot(oh, rstd, preferred_element_type=jnp.float32)
    o = cen * rstd_n * lnw[...] + lnb[...]
    out_ref[...] = jnp.where(o >= 0, o, 0.01 * o)


def _finalize(num2, den2, skip, batch, ln_w, ln_b):
    return pl.pallas_call(
        _fin_body,
        out_shape=[
            jax.ShapeDtypeStruct((N, HID), jnp.float32),
            jax.ShapeDtypeStruct((N, H), jnp.float32),
        ],
    )(num2, den2, skip, batch.reshape(N, 1), ln_w.reshape(1, HID),
      ln_b.reshape(1, HID))


def _pass2_body(ex_hbm, dst_hbm, inv_hbm, att_out, dstv, exb, invv, attb, sem):
    cid = lax.axis_index("c")
    sid = lax.axis_index("s")
    wid = cid * NS + sid
    pltpu.sync_copy(inv_hbm, invv)
    iota = lax.iota(jnp.int32, L)
    hi = lax.shift_right_logical(iota, 3)
    lo = jnp.bitwise_and(iota, 7)

    def c2_body(c, carry):
        base = wid * EPW + c * CH2
        pltpu.sync_copy(dst_hbm.at[pl.ds(base, CH2)], dstv)
        pltpu.sync_copy(ex_hbm.at[pl.ds(base, CH2)], exb)

        @plsc.parallel_loop(0, CH2 * H // L, unroll=4)
        def v_body(i):
            ridx = i * 2 + hi
            dv = plsc.load_gather(dstv, [ridx])
            iv = plsc.load_gather(invv, [dv, lo])
            ev = plsc.load_gather(exb, [ridx, lo])
            plsc.store_scatter(attb, [ridx, lo], ev * iv)
        pltpu.sync_copy(attb, att_out.at[pl.ds(base, CH2)])
        return carry

    lax.fori_loop(0, NCHUNK2, c2_body, 0)


def _pass2(ex, dst, inv_den):
    mesh = plsc.VectorSubcoreMesh(
        core_axis_name="c", subcore_axis_name="s",
        num_cores=NC, num_subcores=NS)
    f = pl.kernel(
        _pass2_body,
        out_type=jax.ShapeDtypeStruct((E, H), jnp.float32),
        mesh=mesh,
        scratch_types=[
            pltpu.VMEM((CH2,), jnp.int32),
            pltpu.VMEM((CH2, H), jnp.float32),
            pltpu.VMEM((N, H), jnp.float32),
            pltpu.VMEM((CH2, H), jnp.float32),
            pltpu.SemaphoreType.DMA,
        ],
        compiler_params=pltpu.CompilerParams(needs_layout_passes=False, use_tc_tiling_on_sc=False),
    )
    return f(ex, dst, inv_den)


def kernel(x, edge_index, batch, Wq, bq, Wk, bk, Wv, bv, Wskip, bskip,
           ln_w, ln_b):
    q, k, v, skip = _proj(x, Wq, bq.reshape(1, HID), Wk, bk.reshape(1, HID),
                          Wv, bv.reshape(1, HID), Wskip, bskip.reshape(1, HID))
    src = edge_index[0]
    dst = edge_index[1]
    num2, den2, ex = _pass1(q, k, v, src, dst)
    out, inv_den = _finalize(num2, den2, skip, batch, ln_w, ln_b)
    att = _pass2(ex, dst, inv_den)
    return out, edge_index, batch, att


# speedup vs baseline: 2.6459x; 2.6459x over previous
import jax
import jax.numpy as jnp
from jax import lax
from jax.experimental import pallas as pl
from jax.experimental.pallas import tpu as pltpu
from jax.experimental.pallas import tpu_sc as plsc

N = 10000
E = 320000
D = 128
H = 8
C = 16
HID = H * C
G = 64

NC = 2
NS = 16
L = 16
NW = NC * NS
EPW = E // NW
CH = 40
NCHUNK = EPW // CH
IB = 50
NBLK = NCHUNK // IB
CH2 = 1000
NCHUNK2 = EPW // CH2
ROWS_PT = 1000
NTILE_IO = N // ROWS_PT


def _proj_body(x_ref, wq, bq, wk, bk, wv, bv, ws, bs, q_ref, k_ref, v_ref,
               s_ref):
    xb = x_ref[...]
    q = (jnp.dot(xb, wq[...], preferred_element_type=jnp.float32) + bq[...])
    q_ref[...] = q * 0.25
    k_ref[...] = jnp.dot(xb, wk[...], preferred_element_type=jnp.float32) + bk[...]
    v_ref[...] = jnp.dot(xb, wv[...], preferred_element_type=jnp.float32) + bv[...]
    s_ref[...] = jnp.dot(xb, ws[...], preferred_element_type=jnp.float32) + bs[...]


_ROWS_BLK = 1000


def _proj(x, Wq, bq, Wk, bk, Wv, bv, Ws, bs):
    full = lambda shape: pl.BlockSpec(shape, lambda i: (0, 0))
    return pl.pallas_call(
        _proj_body,
        grid=(N // _ROWS_BLK,),
        in_specs=[
            pl.BlockSpec((_ROWS_BLK, D), lambda i: (i, 0)),
            full((D, HID)), full((1, HID)),
            full((D, HID)), full((1, HID)),
            full((D, HID)), full((1, HID)),
            full((D, HID)), full((1, HID)),
        ],
        out_specs=[
            pl.BlockSpec((_ROWS_BLK, HID), lambda i: (i, 0)),
            pl.BlockSpec((_ROWS_BLK, HID), lambda i: (i, 0)),
            pl.BlockSpec((_ROWS_BLK, HID), lambda i: (i, 0)),
            pl.BlockSpec((_ROWS_BLK, HID), lambda i: (i, 0)),
        ],
        out_shape=[
            jax.ShapeDtypeStruct((N, HID), jnp.float32),
            jax.ShapeDtypeStruct((N, HID), jnp.float32),
            jax.ShapeDtypeStruct((N, HID), jnp.float32),
            jax.ShapeDtypeStruct((N, HID), jnp.float32),
        ],
    )(x, Wq, bq, Wk, bk, Wv, bv, Ws, bs)


def _pass1_body(q_hbm, k_hbm, v_hbm, src_hbm, dst_hbm, znum, zden,
                num_out, den_out, ex_out,
                src_all, dst_all, krows0, vrows0, qrows0, exbuf0,
                krows1, vrows1, qrows1, exbuf1,
                accn, accd, gsem0, gsem1):
    cid = lax.axis_index("c")
    sid = lax.axis_index("s")
    wid = cid * NS + sid
    r0 = sid * ROWS_PT
    @pl.when(sid < NTILE_IO)
    def _zero():
        pltpu.sync_copy(znum, accn.at[pl.ds(r0, ROWS_PT)])
        pltpu.sync_copy(zden, accd.at[pl.ds(r0, ROWS_PT)])
    plsc.subcore_barrier()
    iota = lax.iota(jnp.int32, L)

    bufs = ((krows0, vrows0, qrows0, exbuf0, gsem0),
            (krows1, vrows1, qrows1, exbuf1, gsem1))

    def stage_a(lc, b):
        krows, vrows, qrows, _, gsem = bufs[b]
        pltpu.async_copy(k_hbm.at[src_all.at[lc]], krows, gsem)
        pltpu.async_copy(v_hbm.at[src_all.at[lc]], vrows, gsem)
        pltpu.async_copy(q_hbm.at[dst_all.at[lc]], qrows, gsem)

    def stage_b(lc, cbase, b):
        krows, vrows, qrows, exbuf, gsem = bufs[b]
        pltpu.make_async_copy(k_hbm.at[src_all.at[lc]], krows, gsem).wait()
        pltpu.make_async_copy(v_hbm.at[src_all.at[lc]], vrows, gsem).wait()
        pltpu.make_async_copy(q_hbm.at[dst_all.at[lc]], qrows, gsem).wait()

        @plsc.parallel_loop(0, CH, unroll=4)
        def edge_body(e):
            ev = jnp.broadcast_to(e, (L,)).astype(jnp.int32)
            lane0 = iota == 0
            for h in range(H):
                qb = qrows[e, pl.ds(h * C, C)]
                kb = krows[e, pl.ds(h * C, C)]
                tot = jnp.sum(qb * kb)
                exh = jnp.exp(jnp.broadcast_to(tot, (L,)))
                vb = vrows[e, pl.ds(h * C, C)]
                vrows[e, pl.ds(h * C, C)] = exh * vb
                plsc.store_scatter(
                    exbuf, [ev, jnp.full((L,), h, jnp.int32)], exh,
                    mask=lane0)
        pltpu.sync_copy(vrows, accn.at[dst_all.at[lc]], add=True)
        pltpu.sync_copy(exbuf, accd.at[dst_all.at[lc]], add=True)
        pltpu.sync_copy(
            exbuf, ex_out.at[pl.ds(wid * EPW + (cbase + lc) * CH, CH)])

    def blk_body(ib, carry):
        cbase = ib * IB
        pltpu.sync_copy(src_hbm.at[wid, pl.ds(cbase, IB)], src_all)
        pltpu.sync_copy(dst_hbm.at[wid, pl.ds(cbase, IB)], dst_all)
        stage_a(0, 0)

        def pair_body(p, carry2):
            lc0 = 2 * p
            stage_a(lc0 + 1, 1)
            stage_b(lc0, cbase, 0)

            @pl.when(lc0 + 2 < IB)
            def _():
                stage_a(lc0 + 2, 0)
            stage_b(lc0 + 1, cbase, 1)
            return carry2

        lax.fori_loop(0, IB // 2, pair_body, 0)
        return carry

    lax.fori_loop(0, NBLK, blk_body, 0)
    plsc.subcore_barrier()

    @pl.when(sid < NTILE_IO)
    def _writeout():
        pltpu.sync_copy(accn.at[pl.ds(r0, ROWS_PT)],
                        num_out.at[cid, pl.ds(r0, ROWS_PT)])
        pltpu.sync_copy(accd.at[pl.ds(r0, ROWS_PT)],
                        den_out.at[cid, pl.ds(r0, ROWS_PT)])


def _pass1(q, k, v, src, dst):
    znum = jnp.zeros((ROWS_PT, HID), jnp.float32)
    zden = jnp.zeros((ROWS_PT, H), jnp.float32)
    mesh = plsc.VectorSubcoreMesh(
        core_axis_name="c", subcore_axis_name="s",
        num_cores=NC, num_subcores=NS)
    f = pl.kernel(
        _pass1_body,
        out_type=(
            jax.ShapeDtypeStruct((NC, N, HID), jnp.float32),
            jax.ShapeDtypeStruct((NC, N, H), jnp.float32),
            jax.ShapeDtypeStruct((E, H), jnp.float32),
        ),
        mesh=mesh,
        scratch_types=[
            pltpu.VMEM((IB, CH), jnp.int32),
            pltpu.VMEM((IB, CH), jnp.int32),
            pltpu.VMEM((CH, HID), jnp.float32),
            pltpu.VMEM((CH, HID), jnp.float32),
            pltpu.VMEM((CH, HID), jnp.float32),
            pltpu.VMEM((CH, H), jnp.float32),
            pltpu.VMEM((CH, HID), jnp.float32),
            pltpu.VMEM((CH, HID), jnp.float32),
            pltpu.VMEM((CH, HID), jnp.float32),
            pltpu.VMEM((CH, H), jnp.float32),
            pltpu.VMEM_SHARED((N, HID), jnp.float32),
            pltpu.VMEM_SHARED((N, H), jnp.float32),
            pltpu.SemaphoreType.DMA,
            pltpu.SemaphoreType.DMA,
        ],
        compiler_params=pltpu.CompilerParams(needs_layout_passes=False, use_tc_tiling_on_sc=False),
    )
    return f(q, k, v, src.reshape(NW, NCHUNK, CH), dst.reshape(NW, NCHUNK, CH),
             znum, zden)


def _fin_body(num_ref, den_ref, skip_ref, batch_ref, lnw, lnb, out_ref, inv_ref):
    num = num_ref[0] + num_ref[1]
    den = den_ref[0] + den_ref[1]
    inv = 1.0 / (den + 1e-16)
    inv_ref[...] = inv
    hh = lax.broadcasted_iota(jnp.int32, (H, HID), 0)
    jj = lax.broadcasted_iota(jnp.int32, (H, HID), 1)
    expand = (jj // C == hh).astype(jnp.float32)
    inv128 = jnp.dot(inv, expand, preferred_element_type=jnp.float32)
    out1 = num * inv128 + skip_ref[...]
    b = batch_ref[...]
    oh = (b == lax.broadcasted_iota(jnp.int32, (N, G), 1)).astype(jnp.float32)
    s = jnp.sum(out1, axis=1, keepdims=True)
    cnt = jnp.sum(oh, axis=0)
    norm = jnp.clip(cnt, 1.0, None) * HID
    dn = (((0,), (0,)), ((), ()))
    sums = lax.dot_general(oh, s, dn, preferred_element_type=jnp.float32)
    mean = sums / norm[:, None]
    mean_n = jnp.dot(oh, mean, preferred_element_type=jnp.float32)
    cen = out1 - mean_n
    s2 = jnp.sum(cen * cen, axis=1, keepdims=True)
    v2 = lax.dot_general(oh, s2, dn, preferred_element_type=jnp.float32)
    rstd = 1.0 / jnp.sqrt(v2 / norm[:, None] + 1e-5)
    rstd_n = jnp.dot(oh, rstd, preferred_element_type=jnp.float32)
    o = cen * rstd_n * lnw[...] + lnb[...]
    out_ref[...] = jnp.where(o >= 0, o, 0.01 * o)


def _finalize(num2, den2, skip, batch, ln_w, ln_b):
    return pl.pallas_call(
        _fin_body,
        out_shape=[
            jax.ShapeDtypeStruct((N, HID), jnp.float32),
            jax.ShapeDtypeStruct((N, H), jnp.float32),
        ],
    )(num2, den2, skip, batch.reshape(N, 1), ln_w.reshape(1, HID),
      ln_b.reshape(1, HID))


def _pass2_body(ex_hbm, dst_hbm, inv_hbm, att_out, dstv, exb, invv, attb, sem):
    cid = lax.axis_index("c")
    sid = lax.axis_index("s")
    wid = cid * NS + sid
    pltpu.sync_copy(inv_hbm, invv)
    iota = lax.iota(jnp.int32, L)
    hi = lax.shift_right_logical(iota, 3)
    lo = jnp.bitwise_and(iota, 7)

    def c2_body(c, carry):
        base = wid * EPW + c * CH2
        pltpu.sync_copy(dst_hbm.at[pl.ds(base, CH2)], dstv)
        pltpu.sync_copy(ex_hbm.at[pl.ds(base, CH2)], exb)

        @plsc.parallel_loop(0, CH2 * H // L, unroll=4)
        def v_body(i):
            ridx = i * 2 + hi
            dv = plsc.load_gather(dstv, [ridx])
            iv = plsc.load_gather(invv, [dv, lo])
            ev = plsc.load_gather(exb, [ridx, lo])
            plsc.store_scatter(attb, [ridx, lo], ev * iv)
        pltpu.sync_copy(attb, att_out.at[pl.ds(base, CH2)])
        return carry

    lax.fori_loop(0, NCHUNK2, c2_body, 0)


def _pass2(ex, dst, inv_den):
    mesh = plsc.VectorSubcoreMesh(
        core_axis_name="c", subcore_axis_name="s",
        num_cores=NC, num_subcores=NS)
    f = pl.kernel(
        _pass2_body,
        out_type=jax.ShapeDtypeStruct((E, H), jnp.float32),
        mesh=mesh,
        scratch_types=[
            pltpu.VMEM((CH2,), jnp.int32),
            pltpu.VMEM((CH2, H), jnp.float32),
            pltpu.VMEM((N, H), jnp.float32),
            pltpu.VMEM((CH2, H), jnp.float32),
            pltpu.SemaphoreType.DMA,
        ],
        compiler_params=pltpu.CompilerParams(needs_layout_passes=False, use_tc_tiling_on_sc=False),
    )
    return f(ex, dst, inv_den)


def kernel(x, edge_index, batch, Wq, bq, Wk, bk, Wv, bv, Wskip, bskip,
           ln_w, ln_b):
    q, k, v, skip = _proj(x, Wq, bq.reshape(1, HID), Wk, bk.reshape(1, HID),
                          Wv, bv.reshape(1, HID), Wskip, bskip.reshape(1, HID))
    src = edge_index[0]
    dst = edge_index[1]
    num2, den2, ex = _pass1(q, k, v, src, dst)
    out, inv_den = _finalize(num2, den2, skip, batch, ln_w, ln_b)
    att = _pass2(ex, dst, inv_den)
    return out, edge_index, batch, att


# speedup vs baseline: 2.6759x; 1.0113x over previous
import jax
import jax.numpy as jnp
from jax import lax
from jax.experimental import pallas as pl
from jax.experimental.pallas import tpu as pltpu
from jax.experimental.pallas import tpu_sc as plsc

N = 10000
E = 320000
D = 128
H = 8
C = 16
HID = H * C
G = 64

NC = 2
NS = 16
L = 16
NW = NC * NS
EPW = E // NW
CH = 40
NCHUNK = EPW // CH
IB = 50
NBLK = NCHUNK // IB
CH2 = 1000
NCHUNK2 = EPW // CH2
ROWS_PT = 1000
NTILE_IO = N // ROWS_PT


def _proj_body(x_ref, wq, bq, wk, bk, wv, bv, ws, bs, q_ref, k_ref, v_ref,
               s_ref):
    xb = x_ref[...]
    q = (jnp.dot(xb, wq[...], preferred_element_type=jnp.float32) + bq[...])
    q_ref[...] = q * 0.25
    k_ref[...] = jnp.dot(xb, wk[...], preferred_element_type=jnp.float32) + bk[...]
    v_ref[...] = jnp.dot(xb, wv[...], preferred_element_type=jnp.float32) + bv[...]
    s_ref[...] = jnp.dot(xb, ws[...], preferred_element_type=jnp.float32) + bs[...]


_ROWS_BLK = 1000


def _proj(x, Wq, bq, Wk, bk, Wv, bv, Ws, bs):
    full = lambda shape: pl.BlockSpec(shape, lambda i: (0, 0))
    return pl.pallas_call(
        _proj_body,
        grid=(N // _ROWS_BLK,),
        in_specs=[
            pl.BlockSpec((_ROWS_BLK, D), lambda i: (i, 0)),
            full((D, HID)), full((1, HID)),
            full((D, HID)), full((1, HID)),
            full((D, HID)), full((1, HID)),
            full((D, HID)), full((1, HID)),
        ],
        out_specs=[
            pl.BlockSpec((_ROWS_BLK, HID), lambda i: (i, 0)),
            pl.BlockSpec((_ROWS_BLK, HID), lambda i: (i, 0)),
            pl.BlockSpec((_ROWS_BLK, HID), lambda i: (i, 0)),
            pl.BlockSpec((_ROWS_BLK, HID), lambda i: (i, 0)),
        ],
        out_shape=[
            jax.ShapeDtypeStruct((N, HID), jnp.float32),
            jax.ShapeDtypeStruct((N, HID), jnp.float32),
            jax.ShapeDtypeStruct((N, HID), jnp.float32),
            jax.ShapeDtypeStruct((N, HID), jnp.float32),
        ],
    )(x, Wq, bq, Wk, bk, Wv, bv, Ws, bs)


def _ei_body(ei_ref, s_ref, d_ref):
    s_ref[...] = ei_ref[0:1, :].reshape(E // D, D)
    d_ref[...] = ei_ref[1:2, :].reshape(E // D, D)


def _edges(edge_index):
    src2, dst2 = pl.pallas_call(
        _ei_body,
        out_shape=[
            jax.ShapeDtypeStruct((E // D, D), jnp.int32),
            jax.ShapeDtypeStruct((E // D, D), jnp.int32),
        ],
    )(edge_index)
    return src2, dst2


def _pass1_body(q_hbm, k_hbm, v_hbm, src_hbm, dst_hbm, znum, zden,
                num_out, den_out, ex_out,
                src_all, dst_all, krows0, vrows0, qrows0, exbuf0,
                krows1, vrows1, qrows1, exbuf1,
                accn, accd, gsem0, gsem1):
    cid = lax.axis_index("c")
    sid = lax.axis_index("s")
    wid = cid * NS + sid
    r0 = sid * ROWS_PT
    @pl.when(sid < NTILE_IO)
    def _zero():
        pltpu.sync_copy(znum, accn.at[pl.ds(r0, ROWS_PT)])
        pltpu.sync_copy(zden, accd.at[pl.ds(r0, ROWS_PT)])
    plsc.subcore_barrier()
    iota = lax.iota(jnp.int32, L)

    bufs = ((krows0, vrows0, qrows0, exbuf0, gsem0),
            (krows1, vrows1, qrows1, exbuf1, gsem1))

    def stage_a(lc, b):
        krows, vrows, qrows, _, gsem = bufs[b]
        pltpu.async_copy(k_hbm.at[src_all.at[lc]], krows, gsem)
        pltpu.async_copy(v_hbm.at[src_all.at[lc]], vrows, gsem)
        pltpu.async_copy(q_hbm.at[dst_all.at[lc]], qrows, gsem)

    def stage_b(lc, cbase, b):
        krows, vrows, qrows, exbuf, gsem = bufs[b]
        pltpu.make_async_copy(k_hbm.at[src_all.at[lc]], krows, gsem).wait()
        pltpu.make_async_copy(v_hbm.at[src_all.at[lc]], vrows, gsem).wait()
        pltpu.make_async_copy(q_hbm.at[dst_all.at[lc]], qrows, gsem).wait()

        @plsc.parallel_loop(0, CH, unroll=4)
        def edge_body(e):
            ev = jnp.broadcast_to(e, (L,)).astype(jnp.int32)
            lane0 = iota == 0
            for h in range(H):
                qb = qrows[e, pl.ds(h * C, C)]
                kb = krows[e, pl.ds(h * C, C)]
                tot = jnp.sum(qb * kb)
                exh = jnp.exp(jnp.broadcast_to(tot, (L,)))
                vb = vrows[e, pl.ds(h * C, C)]
                vrows[e, pl.ds(h * C, C)] = exh * vb
                plsc.store_scatter(
                    exbuf, [ev, jnp.full((L,), h, jnp.int32)], exh,
                    mask=lane0)
        pltpu.sync_copy(vrows, accn.at[dst_all.at[lc]], add=True)
        pltpu.sync_copy(exbuf, accd.at[dst_all.at[lc]], add=True)
        pltpu.sync_copy(
            exbuf, ex_out.at[pl.ds(wid * EPW + (cbase + lc) * CH, CH)])

    def blk_body(ib, carry):
        cbase = ib * IB
        pltpu.sync_copy(src_hbm.at[wid, pl.ds(cbase, IB)], src_all)
        pltpu.sync_copy(dst_hbm.at[wid, pl.ds(cbase, IB)], dst_all)
        stage_a(0, 0)

        def pair_body(p, carry2):
            lc0 = 2 * p
            stage_a(lc0 + 1, 1)
            stage_b(lc0, cbase, 0)

            @pl.when(lc0 + 2 < IB)
            def _():
                stage_a(lc0 + 2, 0)
            stage_b(lc0 + 1, cbase, 1)
            return carry2

        lax.fori_loop(0, IB // 2, pair_body, 0)
        return carry

    lax.fori_loop(0, NBLK, blk_body, 0)
    plsc.subcore_barrier()

    @pl.when(sid < NTILE_IO)
    def _writeout():
        pltpu.sync_copy(accn.at[pl.ds(r0, ROWS_PT)],
                        num_out.at[cid, pl.ds(r0, ROWS_PT)])
        pltpu.sync_copy(accd.at[pl.ds(r0, ROWS_PT)],
                        den_out.at[cid, pl.ds(r0, ROWS_PT)])


def _pass1(q, k, v, src, dst):
    znum = jnp.zeros((ROWS_PT, HID), jnp.float32)
    zden = jnp.zeros((ROWS_PT, H), jnp.float32)
    mesh = plsc.VectorSubcoreMesh(
        core_axis_name="c", subcore_axis_name="s",
        num_cores=NC, num_subcores=NS)
    f = pl.kernel(
        _pass1_body,
        out_type=(
            jax.ShapeDtypeStruct((NC, N, HID), jnp.float32),
            jax.ShapeDtypeStruct((NC, N, H), jnp.float32),
            jax.ShapeDtypeStruct((E, H), jnp.float32),
        ),
        mesh=mesh,
        scratch_types=[
            pltpu.VMEM((IB, CH), jnp.int32),
            pltpu.VMEM((IB, CH), jnp.int32),
            pltpu.VMEM((CH, HID), jnp.float32),
            pltpu.VMEM((CH, HID), jnp.float32),
            pltpu.VMEM((CH, HID), jnp.float32),
            pltpu.VMEM((CH, H), jnp.float32),
            pltpu.VMEM((CH, HID), jnp.float32),
            pltpu.VMEM((CH, HID), jnp.float32),
            pltpu.VMEM((CH, HID), jnp.float32),
            pltpu.VMEM((CH, H), jnp.float32),
            pltpu.VMEM_SHARED((N, HID), jnp.float32),
            pltpu.VMEM_SHARED((N, H), jnp.float32),
            pltpu.SemaphoreType.DMA,
            pltpu.SemaphoreType.DMA,
        ],
        compiler_params=pltpu.CompilerParams(needs_layout_passes=False, use_tc_tiling_on_sc=False),
    )
    return f(q, k, v, src.reshape(NW, NCHUNK, CH), dst.reshape(NW, NCHUNK, CH),
             znum, zden)


def _fin_body(num_ref, den_ref, skip_ref, batch_ref, lnw, lnb, out_ref, inv_ref):
    num = num_ref[0] + num_ref[1]
    den = den_ref[0] + den_ref[1]
    inv = 1.0 / (den + 1e-16)
    inv_ref[...] = inv
    hh = lax.broadcasted_iota(jnp.int32, (H, HID), 0)
    jj = lax.broadcasted_iota(jnp.int32, (H, HID), 1)
    expand = (jj // C == hh).astype(jnp.float32)
    inv128 = jnp.dot(inv, expand, preferred_element_type=jnp.float32)
    out1 = num * inv128 + skip_ref[...]
    b = batch_ref[...]
    oh = (b == lax.broadcasted_iota(jnp.int32, (N, G), 1)).astype(jnp.float32)
    s = jnp.sum(out1, axis=1, keepdims=True)
    cnt = jnp.sum(oh, axis=0)
    norm = jnp.clip(cnt, 1.0, None) * HID
    dn = (((0,), (0,)), ((), ()))
    sums = lax.dot_general(oh, s, dn, preferred_element_type=jnp.float32)
    mean = sums / norm[:, None]
    mean_n = jnp.dot(oh, mean, preferred_element_type=jnp.float32)
    cen = out1 - mean_n
    s2 = jnp.sum(cen * cen, axis=1, keepdims=True)
    v2 = lax.dot_general(oh, s2, dn, preferred_element_type=jnp.float32)
    rstd = 1.0 / jnp.sqrt(v2 / norm[:, None] + 1e-5)
    rstd_n = jnp.dot(oh, rstd, preferred_element_type=jnp.float32)
    o = cen * rstd_n * lnw[...] + lnb[...]
    out_ref[...] = jnp.where(o >= 0, o, 0.01 * o)


def _finalize(num2, den2, skip, batch, ln_w, ln_b):
    return pl.pallas_call(
        _fin_body,
        out_shape=[
            jax.ShapeDtypeStruct((N, HID), jnp.float32),
            jax.ShapeDtypeStruct((N, H), jnp.float32),
        ],
    )(num2, den2, skip, batch.reshape(N, 1), ln_w.reshape(1, HID),
      ln_b.reshape(1, HID))


def _pass2_body(ex_hbm, dst_hbm, inv_hbm, att_out, dstv, exb, invv, attb, sem):
    cid = lax.axis_index("c")
    sid = lax.axis_index("s")
    wid = cid * NS + sid
    pltpu.sync_copy(inv_hbm, invv)
    iota = lax.iota(jnp.int32, L)
    hi = lax.shift_right_logical(iota, 3)
    lo = jnp.bitwise_and(iota, 7)

    def c2_body(c, carry):
        base = wid * EPW + c * CH2
        pltpu.sync_copy(dst_hbm.at[pl.ds(base, CH2)], dstv)
        pltpu.sync_copy(ex_hbm.at[pl.ds(base, CH2)], exb)

        @plsc.parallel_loop(0, CH2 * H // L, unroll=4)
        def v_body(i):
            ridx = i * 2 + hi
            dv = plsc.load_gather(dstv, [ridx])
            iv = plsc.load_gather(invv, [dv, lo])
            ev = plsc.load_gather(exb, [ridx, lo])
            plsc.store_scatter(attb, [ridx, lo], ev * iv)
        pltpu.sync_copy(attb, att_out.at[pl.ds(base, CH2)])
        return carry

    lax.fori_loop(0, NCHUNK2, c2_body, 0)


def _pass2(ex, dst, inv_den):
    mesh = plsc.VectorSubcoreMesh(
        core_axis_name="c", subcore_axis_name="s",
        num_cores=NC, num_subcores=NS)
    f = pl.kernel(
        _pass2_body,
        out_type=jax.ShapeDtypeStruct((E, H), jnp.float32),
        mesh=mesh,
        scratch_types=[
            pltpu.VMEM((CH2,), jnp.int32),
            pltpu.VMEM((CH2, H), jnp.float32),
            pltpu.VMEM((N, H), jnp.float32),
            pltpu.VMEM((CH2, H), jnp.float32),
            pltpu.SemaphoreType.DMA,
        ],
        compiler_params=pltpu.CompilerParams(needs_layout_passes=False, use_tc_tiling_on_sc=False),
    )
    return f(ex, dst, inv_den)


def kernel(x, edge_index, batch, Wq, bq, Wk, bk, Wv, bv, Wskip, bskip,
           ln_w, ln_b):
    q, k, v, skip = _proj(x, Wq, bq.reshape(1, HID), Wk, bk.reshape(1, HID),
                          Wv, bv.reshape(1, HID), Wskip, bskip.reshape(1, HID))
    src2, dst2 = _edges(edge_index)
    src = src2.reshape(E)
    dst = dst2.reshape(E)
    num2, den2, ex = _pass1(q, k, v, src, dst)
    out, inv_den = _finalize(num2, den2, skip, batch, ln_w, ln_b)
    att = _pass2(ex, dst, inv_den)
    return out, edge_index, batch, att
